# baseline (speedup 1.0000x reference)
import functools

import jax
import jax.numpy as jnp
from jax import lax
from jax.experimental import pallas as pl
from jax.experimental.pallas import tpu as pltpu
from jax.experimental.pallas import tpu_sc as plsc

_NC = 2
_NS = 16
_CH = 128
_W = 16


def kernel(user_idx, year, num_ratings, user_table, year_table,
           rating_table, year_bounds, rating_bounds):
    B = user_idx.shape[0]
    V = user_table.shape[0]
    E = user_table.shape[1]
    C = 3 * E
    nbnd = year_bounds.shape[0]
    nbins = year_table.shape[0]
    NW = _NC * _NS
    bpw = B // NW
    nch = bpw // _CH
    nwin = (V * E) // _W
    mesh = plsc.VectorSubcoreMesh(core_axis_name="c", subcore_axis_name="s")

    zero = lax.optimization_barrier(year_bounds[0] * 0.0)
    utab_win = user_table.reshape(-1)[:nwin * _W].reshape(nwin, _W) + zero
    ulast = user_table[V - 1]
    ytab_pad = jnp.pad(year_table, ((0, 0), (0, _W - E)))
    rtab_pad = jnp.pad(rating_table, ((0, 0), (0, _W - E)))

    @functools.partial(
        pl.kernel,
        mesh=mesh,
        out_type=jax.ShapeDtypeStruct((B * C,), jnp.float32),
        compiler_params=pltpu.CompilerParams(use_tc_tiling_on_sc=False),
        scratch_types=[
            pltpu.VMEM((nch, _CH), jnp.int32),
            pltpu.VMEM((nch, _CH), jnp.float32),
            pltpu.VMEM((nch, _CH), jnp.float32),
            pltpu.VMEM((32,), jnp.float32),
            pltpu.VMEM((32,), jnp.float32),
            pltpu.VMEM((nch, 2, _CH), jnp.int32),
            pltpu.VMEM((nch, _CH), jnp.float32),
            pltpu.VMEM((nch, _CH), jnp.float32),
            pltpu.VMEM((nch, _CH), jnp.int32),
            pltpu.VMEM((nch, _CH), jnp.int32),
            pltpu.VMEM((nch, _CH, _W), jnp.float32),
            pltpu.VMEM((nch, _CH, _W), jnp.float32),
            pltpu.VMEM((nch, _CH, _W), jnp.float32),
            pltpu.VMEM((nch, _CH, _W), jnp.float32),
            pltpu.VMEM((nch, _CH * C + _W), jnp.float32),
            pltpu.VMEM((16,), jnp.float32),
            pltpu.SemaphoreType.DMA,
        ],
    )
    def sc_kernel(uidx_h, year_h, rate_h, uwin_h, ulast_h, ytab_h, rtab_h,
                  ybnd_h, rbnd_h, out_h,
                  idx_v, yv_v, rv_v, ybnd_v, rbnd_v, win_v, roff_v, last_v,
                  ybin_v, rbin_v, w0blk, w1blk, yblk, rblk, outblk,
                  lrow_v, sem):
        wid = lax.axis_index("s") * _NC + lax.axis_index("c")
        base = wid * bpw

        for j in range(nch):
            pltpu.sync_copy(uidx_h.at[pl.ds(base + j * _CH, _CH)], idx_v.at[j])
            pltpu.sync_copy(year_h.at[pl.ds(base + j * _CH, _CH)], yv_v.at[j])
            pltpu.sync_copy(rate_h.at[pl.ds(base + j * _CH, _CH)], rv_v.at[j])
        pltpu.sync_copy(ybnd_h, ybnd_v.at[pl.ds(0, nbnd)])
        pltpu.sync_copy(rbnd_h, rbnd_v.at[pl.ds(0, nbnd)])
        pltpu.sync_copy(ulast_h, lrow_v.at[pl.ds(0, E)])

        lane = lax.iota(jnp.int32, 16)
        yb0 = ybnd_v[pl.ds(0, 16)]
        yb1 = ybnd_v[pl.ds(16, 16)]
        rb0 = rbnd_v[pl.ds(0, 16)]
        rb1 = rbnd_v[pl.ds(16, 16)]

        gdn = lax.GatherDimensionNumbers(
            offset_dims=(), collapsed_slice_dims=(0,), start_index_map=(0,))

        def bcast(vec, idx):
            return lax.gather(vec, idx.reshape(16, 1), gdn, (1,),
                              mode=lax.GatherScatterMode.PROMISE_IN_BOUNDS)

        def rank(b0, b1, v):
            pos = jnp.zeros((16,), jnp.int32)
            for sz in (16, 8, 4, 2, 1):
                nxt = pos + sz
                probe = jnp.minimum(nxt - 1, nbnd - 1)
                g0 = bcast(b0, jnp.minimum(probe, 15))
                g1 = bcast(b1, jnp.clip(probe - 16, 0, 15))
                bv = jnp.where(probe < 16, g0, g1)
                take = (nxt <= nbnd) & (bv <= v)
                pos = jnp.where(take, nxt, pos)
            return jnp.minimum(pos, nbins - 1)

        cps = []
        for j in range(nch):
            def cbody(i8, carry, j=j):
                off = i8 * 16
                uix = idx_v[j, pl.ds(off, 16)]
                w0 = uix * E
                k0 = jnp.minimum(lax.shift_right_logical(w0, 4), nwin - 1)
                win_v[j, 0, pl.ds(off, 16)] = k0
                win_v[j, 1, pl.ds(off, 16)] = jnp.minimum(k0 + 1, nwin - 1)
                roff_v[j, pl.ds(off, 16)] = (w0 - k0 * _W).astype(jnp.float32)
                lastf = (uix - (V - 2)).astype(jnp.float32)
                last_v[j, pl.ds(off, 16)] = jnp.clip(lastf, 0.0, 1.0)
                ybin_v[j, pl.ds(off, 16)] = rank(yb0, yb1,
                                                 yv_v[j, pl.ds(off, 16)])
                rbin_v[j, pl.ds(off, 16)] = rank(rb0, rb1,
                                                 rv_v[j, pl.ds(off, 16)])
                return carry
            lax.fori_loop(0, _CH // 16, cbody, 0)
            cps.append(pltpu.async_copy(
                uwin_h.at[win_v.at[j, 0]], w0blk.at[j], sem))
            cps.append(pltpu.async_copy(
                uwin_h.at[win_v.at[j, 1]], w1blk.at[j], sem))
            cps.append(pltpu.async_copy(
                ytab_h.at[ybin_v.at[j]], yblk.at[j], sem))
            cps.append(pltpu.async_copy(
                rtab_h.at[rbin_v.at[j]], rblk.at[j], sem))
        for c in cps:
            c.wait()

        lrow = lrow_v[pl.ds(0, 16)]

        for j in range(nch):
            def abody(i8, carry, j=j):
                off = i8 * 16
                r16 = roff_v[j, pl.ds(off, 16)]
                l16 = last_v[j, pl.ds(off, 16)]
                for l in range(16):
                    e = off + l
                    lsel = jnp.full((16,), l, jnp.int32)
                    rbi = bcast(r16, lsel).astype(jnp.int32)
                    s = lane + rbi
                    a = w0blk[j, e]
                    b = w1blk[j, e]
                    ga = bcast(a, jnp.minimum(s, 15))
                    gb = bcast(b, jnp.clip(s - 16, 0, 15))
                    u = jnp.where(s < 16, ga, gb)
                    lb = bcast(l16, lsel)
                    u = u + lb * (lrow - u)
                    e30 = e * C
                    outblk[j, pl.ds(e30, 16)] = u
                    outblk[j, pl.ds(e30 + E, 16)] = yblk[j, e]
                    outblk[j, pl.ds(e30 + 2 * E, 16)] = rblk[j, e]
                return carry
            lax.fori_loop(0, _CH // 16, abody, 0)

        for j in range(nch):
            pltpu.sync_copy(outblk.at[j, pl.ds(0, _CH * C)],
                            out_h.at[pl.ds((base + j * _CH) * C, _CH * C)])

    out = sc_kernel(user_idx, year, num_ratings, utab_win, ulast,
                    ytab_pad, rtab_pad, year_bounds, rating_bounds)
    return out.reshape(B, C) + zero


# speedup vs baseline: 1.0558x; 1.0558x over previous
import functools

import jax
import jax.numpy as jnp
from jax import lax
from jax.experimental import pallas as pl
from jax.experimental.pallas import tpu as pltpu
from jax.experimental.pallas import tpu_sc as plsc

_NC = 2
_NS = 16
_CH = 128
_W = 16


def kernel(user_idx, year, num_ratings, user_table, year_table,
           rating_table, year_bounds, rating_bounds):
    B = user_idx.shape[0]
    V = user_table.shape[0]
    E = user_table.shape[1]
    C = 3 * E
    nbnd = year_bounds.shape[0]
    nbins = year_table.shape[0]
    NW = _NC * _NS
    bpw = B // NW
    nch = bpw // _CH
    nwin = (V * E) // _W
    mesh = plsc.VectorSubcoreMesh(core_axis_name="c", subcore_axis_name="s")

    utab_win = user_table.reshape(-1)[:nwin * _W].reshape(nwin, _W)
    ulast = user_table[V - 1]
    ytab_pad = jnp.pad(year_table, ((0, 0), (0, _W - E)))
    rtab_pad = jnp.pad(rating_table, ((0, 0), (0, _W - E)))

    @functools.partial(
        pl.kernel,
        mesh=mesh,
        out_type=jax.ShapeDtypeStruct((B * C,), jnp.float32),
        compiler_params=pltpu.CompilerParams(use_tc_tiling_on_sc=False),
        scratch_types=[
            pltpu.VMEM((nch, _CH), jnp.int32),
            pltpu.VMEM((nch, _CH), jnp.float32),
            pltpu.VMEM((nch, _CH), jnp.float32),
            pltpu.VMEM((32,), jnp.float32),
            pltpu.VMEM((32,), jnp.float32),
            pltpu.VMEM((nch, 2, _CH), jnp.int32),
            pltpu.VMEM((nch, _CH), jnp.float32),
            pltpu.VMEM((nch, _CH), jnp.float32),
            pltpu.VMEM((nch, _CH), jnp.int32),
            pltpu.VMEM((nch, _CH), jnp.int32),
            pltpu.VMEM((nch, _CH, _W), jnp.float32),
            pltpu.VMEM((nch, _CH, _W), jnp.float32),
            pltpu.VMEM((nch, _CH, _W), jnp.float32),
            pltpu.VMEM((nch, _CH, _W), jnp.float32),
            pltpu.VMEM((nch, _CH * C + _W), jnp.float32),
            pltpu.VMEM((16,), jnp.float32),
            pltpu.SemaphoreType.DMA,
        ],
    )
    def sc_kernel(uidx_h, year_h, rate_h, uwin_h, ulast_h, ytab_h, rtab_h,
                  ybnd_h, rbnd_h, out_h,
                  idx_v, yv_v, rv_v, ybnd_v, rbnd_v, win_v, roff_v, last_v,
                  ybin_v, rbin_v, w0blk, w1blk, yblk, rblk, outblk,
                  lrow_v, sem):
        wid = lax.axis_index("s") * _NC + lax.axis_index("c")
        base = wid * bpw

        for j in range(nch):
            pltpu.sync_copy(uidx_h.at[pl.ds(base + j * _CH, _CH)], idx_v.at[j])
            pltpu.sync_copy(year_h.at[pl.ds(base + j * _CH, _CH)], yv_v.at[j])
            pltpu.sync_copy(rate_h.at[pl.ds(base + j * _CH, _CH)], rv_v.at[j])
        pltpu.sync_copy(ybnd_h, ybnd_v.at[pl.ds(0, nbnd)])
        pltpu.sync_copy(rbnd_h, rbnd_v.at[pl.ds(0, nbnd)])
        pltpu.sync_copy(ulast_h, lrow_v.at[pl.ds(0, E)])

        lane = lax.iota(jnp.int32, 16)
        yb0 = ybnd_v[pl.ds(0, 16)]
        yb1 = ybnd_v[pl.ds(16, 16)]
        rb0 = rbnd_v[pl.ds(0, 16)]
        rb1 = rbnd_v[pl.ds(16, 16)]

        gdn = lax.GatherDimensionNumbers(
            offset_dims=(), collapsed_slice_dims=(0,), start_index_map=(0,))

        def bcast(vec, idx):
            return lax.gather(vec, idx.reshape(16, 1), gdn, (1,),
                              mode=lax.GatherScatterMode.PROMISE_IN_BOUNDS)

        def rank(b0, b1, v):
            pos = jnp.zeros((16,), jnp.int32)
            for sz in (16, 8, 4, 2, 1):
                nxt = pos + sz
                probe = jnp.minimum(nxt - 1, nbnd - 1)
                g0 = bcast(b0, jnp.minimum(probe, 15))
                g1 = bcast(b1, jnp.clip(probe - 16, 0, 15))
                bv = jnp.where(probe < 16, g0, g1)
                take = (nxt <= nbnd) & (bv <= v)
                pos = jnp.where(take, nxt, pos)
            return jnp.minimum(pos, nbins - 1)

        cps = []
        for j in range(nch):
            def cbody(i8, carry, j=j):
                off = i8 * 16
                uix = idx_v[j, pl.ds(off, 16)]
                w0 = uix * E
                k0 = jnp.minimum(lax.shift_right_logical(w0, 4), nwin - 1)
                win_v[j, 0, pl.ds(off, 16)] = k0
                win_v[j, 1, pl.ds(off, 16)] = jnp.minimum(k0 + 1, nwin - 1)
                roff_v[j, pl.ds(off, 16)] = (w0 - k0 * _W).astype(jnp.float32)
                lastf = (uix - (V - 2)).astype(jnp.float32)
                last_v[j, pl.ds(off, 16)] = jnp.clip(lastf, 0.0, 1.0)
                ybin_v[j, pl.ds(off, 16)] = rank(yb0, yb1,
                                                 yv_v[j, pl.ds(off, 16)])
                rbin_v[j, pl.ds(off, 16)] = rank(rb0, rb1,
                                                 rv_v[j, pl.ds(off, 16)])
                return carry
            lax.fori_loop(0, _CH // 16, cbody, 0)
            cps.append(pltpu.async_copy(
                uwin_h.at[win_v.at[j, 0]], w0blk.at[j], sem))
            cps.append(pltpu.async_copy(
                uwin_h.at[win_v.at[j, 1]], w1blk.at[j], sem))
            cps.append(pltpu.async_copy(
                ytab_h.at[ybin_v.at[j]], yblk.at[j], sem))
            cps.append(pltpu.async_copy(
                rtab_h.at[rbin_v.at[j]], rblk.at[j], sem))
        for c in cps:
            c.wait()

        lrow = lrow_v[pl.ds(0, 16)]

        for j in range(nch):
            def abody(i8, carry, j=j):
                off = i8 * 16
                r16 = roff_v[j, pl.ds(off, 16)]
                l16 = last_v[j, pl.ds(off, 16)]
                for l in range(16):
                    e = off + l
                    lsel = jnp.full((16,), l, jnp.int32)
                    rbi = bcast(r16, lsel).astype(jnp.int32)
                    s = lane + rbi
                    a = w0blk[j, e]
                    b = w1blk[j, e]
                    ga = bcast(a, jnp.minimum(s, 15))
                    gb = bcast(b, jnp.clip(s - 16, 0, 15))
                    u = jnp.where(s < 16, ga, gb)
                    lb = bcast(l16, lsel)
                    u = u + lb * (lrow - u)
                    e30 = e * C
                    outblk[j, pl.ds(e30, 16)] = u
                    outblk[j, pl.ds(e30 + E, 16)] = yblk[j, e]
                    outblk[j, pl.ds(e30 + 2 * E, 16)] = rblk[j, e]
                return carry
            lax.fori_loop(0, _CH // 16, abody, 0)

        for j in range(nch):
            pltpu.sync_copy(outblk.at[j, pl.ds(0, _CH * C)],
                            out_h.at[pl.ds((base + j * _CH) * C, _CH * C)])

    out = sc_kernel(user_idx, year, num_ratings, utab_win, ulast,
                    ytab_pad, rtab_pad, year_bounds, rating_bounds)
    return out.reshape(B, C)
